# baseline (speedup 1.0000x reference)
import functools

import jax
import jax.numpy as jnp
from jax import lax
from jax.experimental import pallas as pl
from jax.experimental.pallas import tpu as pltpu
from jax.experimental.pallas import tpu_sc as plsc

N, D, K = 16384, 128, 1024
BLK = 4096
IDX_CHUNK = 128


def _assign_body(z_ref, c_ref, zn_ref, cn_ref, idx_ref):
    c = c_ref[...]
    z = z_ref[...]
    s = lax.dot_general(
        c * 2.0, z, (((1,), (1,)), ((), ())),
        preferred_element_type=jnp.float32,
    )
    s = (s - zn_ref[...]) - cn_ref[...]
    m = jnp.max(s, axis=0, keepdims=True)
    ids = lax.broadcasted_iota(jnp.int32, (K, BLK), 0)
    first = jnp.min(jnp.where(s == m, ids, K), axis=0)
    idx_ref[0, 0, :] = first.astype(jnp.int32)


def _assign(z, centroids, zn, cn):
    grid = N // BLK
    return pl.pallas_call(
        _assign_body,
        grid=(grid,),
        in_specs=[
            pl.BlockSpec((BLK, D), lambda i: (i, 0)),
            pl.BlockSpec((K, D), lambda i: (0, 0)),
            pl.BlockSpec((1, BLK), lambda i: (0, i)),
            pl.BlockSpec((K, 1), lambda i: (0, 0)),
        ],
        out_specs=pl.BlockSpec((1, 1, BLK), lambda i: (i, 0, 0)),
        out_shape=jax.ShapeDtypeStruct((grid, 1, BLK), jnp.int32),
    )(z, centroids, zn, cn)


def _sc_gather(centroids, idx):
    try:
        info = plsc.get_sparse_core_info()
        nc, ns = info.num_cores, info.num_subcores
    except Exception:
        nc, ns = 2, 16
    nw = nc * ns
    b_per_w = N // nw
    n_chunks = b_per_w // IDX_CHUNK
    idx3 = idx.reshape(nw, n_chunks, IDX_CHUNK)

    @functools.partial(
        pl.kernel,
        out_type=jax.ShapeDtypeStruct((N, D), jnp.float32),
        mesh=plsc.VectorSubcoreMesh(core_axis_name="c", subcore_axis_name="s"),
        scratch_types=[
            pltpu.VMEM((n_chunks, IDX_CHUNK), jnp.int32),
            pltpu.VMEM((b_per_w, D), jnp.float32),
            pltpu.VMEM_SHARED((K, D), jnp.float32),
            pltpu.SemaphoreType.DMA,
            pltpu.SemaphoreType.DMA,
        ],
    )
    def gather(c_hbm, idx_hbm, out_hbm, idx_v, rows_v, c_sp, sem, wsem):
        sid = lax.axis_index("s")
        wid = sid * nc + lax.axis_index("c")
        base = wid * b_per_w
        rows_per_tile = K // ns
        stage = pltpu.async_copy(
            c_hbm.at[pl.ds(sid * rows_per_tile, rows_per_tile)],
            c_sp.at[pl.ds(sid * rows_per_tile, rows_per_tile)],
            wsem,
        )
        pltpu.sync_copy(idx_hbm.at[wid], idx_v)
        stage.wait()
        plsc.subcore_barrier()
        copies = [
            pltpu.async_copy(
                c_sp.at[idx_v.at[j]],
                rows_v.at[pl.ds(j * IDX_CHUNK, IDX_CHUNK)],
                sem,
            )
            for j in range(n_chunks)
        ]
        writes = []
        for j, cp in enumerate(copies):
            cp.wait()
            writes.append(pltpu.async_copy(
                rows_v.at[pl.ds(j * IDX_CHUNK, IDX_CHUNK)],
                out_hbm.at[pl.ds(base + j * IDX_CHUNK, IDX_CHUNK)],
                wsem,
            ))
        for wr in writes:
            wr.wait()

    return gather(centroids, idx3)


def kernel(z, centroids):
    zn = (z ** 2).sum(axis=1)[None, :]
    cn = (centroids ** 2).sum(axis=1)[:, None]
    idx = _assign(z, centroids, zn, cn)
    return _sc_gather(centroids, idx.reshape(N))


# speedup vs baseline: 1.1609x; 1.1609x over previous
import functools

import jax
import jax.numpy as jnp
from jax import lax
from jax.experimental import pallas as pl
from jax.experimental.pallas import tpu as pltpu
from jax.experimental.pallas import tpu_sc as plsc

N, D, K = 16384, 128, 1024
BLK = 4096
IDX_CHUNK = 128


def _assign_body(z_ref, c_ref, zn_ref, cn_ref, idx_ref):
    c = c_ref[...]
    z = z_ref[...]
    c2 = c * 2.0
    zn = zn_ref[...]
    cn = cn_ref[...]
    KB = 128
    val = None
    gid = None
    for kb in range(K // KB):
        a = lax.dot_general(
            c2[kb * KB:(kb + 1) * KB], z, (((1,), (1,)), ((), ())),
            preferred_element_type=jnp.float32,
        )
        sk = (a - zn) - cn[kb * KB:(kb + 1) * KB]
        for r in range(KB // 8):
            cur = sk[r * 8:(r + 1) * 8]
            g = kb * (KB // 8) + r
            if val is None:
                val = cur
                gid = jnp.zeros(cur.shape, jnp.int32)
            else:
                p = cur > val
                val = jnp.maximum(cur, val)
                gid = jnp.where(p, g, gid)
    k8 = gid * 8 + lax.broadcasted_iota(jnp.int32, val.shape, 0)
    m8 = jnp.max(val, axis=0, keepdims=True)
    first = jnp.min(jnp.where(val == m8, k8, K), axis=0)
    idx_ref[0, 0, :] = first.astype(jnp.int32)


def _assign(z, centroids, zn, cn):
    grid = N // BLK
    return pl.pallas_call(
        _assign_body,
        grid=(grid,),
        in_specs=[
            pl.BlockSpec((BLK, D), lambda i: (i, 0)),
            pl.BlockSpec((K, D), lambda i: (0, 0)),
            pl.BlockSpec((1, BLK), lambda i: (0, i)),
            pl.BlockSpec((K, 1), lambda i: (0, 0)),
        ],
        out_specs=pl.BlockSpec((1, 1, BLK), lambda i: (i, 0, 0)),
        out_shape=jax.ShapeDtypeStruct((grid, 1, BLK), jnp.int32),
    )(z, centroids, zn, cn)


def _sc_gather(centroids, idx):
    try:
        info = plsc.get_sparse_core_info()
        nc, ns = info.num_cores, info.num_subcores
    except Exception:
        nc, ns = 2, 16
    nw = nc * ns
    b_per_w = N // nw
    n_chunks = b_per_w // IDX_CHUNK
    idx3 = idx.reshape(nw, n_chunks, IDX_CHUNK)

    @functools.partial(
        pl.kernel,
        out_type=jax.ShapeDtypeStruct((N, D), jnp.float32),
        mesh=plsc.VectorSubcoreMesh(core_axis_name="c", subcore_axis_name="s"),
        scratch_types=[
            pltpu.VMEM((n_chunks, IDX_CHUNK), jnp.int32),
            pltpu.VMEM((b_per_w, D), jnp.float32),
            pltpu.VMEM_SHARED((K, D), jnp.float32),
            pltpu.SemaphoreType.DMA,
            pltpu.SemaphoreType.DMA,
        ],
    )
    def gather(c_hbm, idx_hbm, out_hbm, idx_v, rows_v, c_sp, sem, wsem):
        sid = lax.axis_index("s")
        wid = sid * nc + lax.axis_index("c")
        base = wid * b_per_w
        rows_per_tile = K // ns
        stage = pltpu.async_copy(
            c_hbm.at[pl.ds(sid * rows_per_tile, rows_per_tile)],
            c_sp.at[pl.ds(sid * rows_per_tile, rows_per_tile)],
            wsem,
        )
        pltpu.sync_copy(idx_hbm.at[wid], idx_v)
        stage.wait()
        plsc.subcore_barrier()
        copies = [
            pltpu.async_copy(
                c_sp.at[idx_v.at[j]],
                rows_v.at[pl.ds(j * IDX_CHUNK, IDX_CHUNK)],
                sem,
            )
            for j in range(n_chunks)
        ]
        writes = []
        for j, cp in enumerate(copies):
            cp.wait()
            writes.append(pltpu.async_copy(
                rows_v.at[pl.ds(j * IDX_CHUNK, IDX_CHUNK)],
                out_hbm.at[pl.ds(base + j * IDX_CHUNK, IDX_CHUNK)],
                wsem,
            ))
        for wr in writes:
            wr.wait()

    return gather(centroids, idx3)


def kernel(z, centroids):
    zn = (z ** 2).sum(axis=1)[None, :]
    cn = (centroids ** 2).sum(axis=1)[:, None]
    idx = _assign(z, centroids, zn, cn)
    return _sc_gather(centroids, idx.reshape(N))
